# baseline (speedup 1.0000x reference)
import functools

import jax
import jax.numpy as jnp
from jax import lax
from jax.experimental import pallas as pl
from jax.experimental.pallas import tpu as pltpu
from jax.experimental.pallas import tpu_sc as plsc

NA, T, M, H, W = 512, 20, 4, 1024, 1024
R = 12
KW = 2 * R + 1
NT = NA * T
NC, NS, L = 2, 16, 16
NWORK = NC * NS
PPW = NT // NWORK
NG = PPW // L
NCH = 3
NROW = KW * NCH
TROWS = M * H * W // L
BIG = 1e9
ACC0 = 1e12


def _body(table, trajf, vehf, mapi, dxa, out,
          rows_v, traj_v, veh_v, map_v, dx_v, out_v, grp_v, sem0, sem1):
    wid = lax.axis_index("s") * NC + lax.axis_index("c")
    iota = lax.iota(jnp.int32, L)

    pltpu.sync_copy(trajf.at[pl.ds(wid * (PPW * 4), PPW * 4)], traj_v)
    pltpu.sync_copy(vehf.at[:], veh_v)
    pltpu.sync_copy(mapi.at[:], map_v)
    pltpu.sync_copy(dxa.at[:], dx_v)
    dxv = dx_v[...]

    def point_params(g):
        lp = g * L + iota
        px = plsc.load_gather(traj_v, [lp * 4])
        py = plsc.load_gather(traj_v, [lp * 4 + 1])
        gp = wid * PPW + lp
        aid = gp // T
        ln = plsc.load_gather(veh_v, [aid * 2])
        wd = plsc.load_gather(veh_v, [aid * 2 + 1])
        mapf = plsc.load_gather(map_v, [aid])
        pd2 = (ln * ln + wd * wd) * 0.25

        pixx = (px / dxv).astype(jnp.int32)
        pixy = (py / dxv).astype(jnp.int32)
        cx0 = pixx - R
        chunk = cx0 >> 4
        s = cx0 & 15
        y0 = pixy - R
        mb = mapf * (H * W // L)
        cps = []
        for c in range(NCH):
            ch = chunk + c
            cps.append(mb + ((ch >> 3) << 6) + (ch & 7))
        fx = px - (pixx.astype(jnp.float32) + 0.5) * dxv
        fy = py - (pixy.astype(jnp.float32) + 0.5) * dxv
        ax = fx + (s + R).astype(jnp.float32) * dxv
        ay = fy + jnp.float32(R) * dxv
        return (y0, cps), ax, ay, pd2

    def fire_group(g, par, base):
        off = par * NROW * L

        y0, cps = base

        def fire(oy, _):
            y = y0 + oy
            ypart = ((y >> 3) << 9) + ((y & 7) << 3)
            for c in range(NCH):
                k = oy * NCH + c
                pltpu.async_copy(table.at[cps[c] + ypart],
                                 rows_v.at[pl.ds(off + k * L, L)],
                                 sem0 if par == 0 else sem1)
            return 0
        lax.fori_loop(0, KW, fire, 0)

    def drain_group(par):
        off = par * NROW * L
        pltpu.make_async_copy(table.at[pl.ds(0, NROW * L)],
                              rows_v.at[pl.ds(off, NROW * L)],
                              sem0 if par == 0 else sem1).wait()

    base0, _, _, _ = point_params(0)
    fire_group(0, 0, base0)

    def group(g, carry):
        par = lax.rem(g, 2)
        base, ax, ay, pd2 = point_params(g)

        @pl.when(g + 1 < NG)
        def _():
            nbase, _, _, _ = point_params(g + 1)
            @pl.when(par == 0)
            def _():
                fire_group(g + 1, 1, nbase)

            @pl.when(par == 1)
            def _():
                fire_group(g + 1, 0, nbase)

        @pl.when(par == 0)
        def _():
            drain_group(0)

        @pl.when(par == 1)
        def _():
            drain_group(1)

        off = par * (NROW * L)
        iota_f = iota.astype(jnp.float32)
        grp_v[pl.ds(0, L)] = ax
        grp_v[pl.ds(L, L)] = ay

        def pbody(p, accg):
            pidx = jnp.full((L,), p, jnp.int32)
            axp = plsc.load_gather(grp_v, [pidx])
            ayp = plsc.load_gather(grp_v, [pidx + L])
            xs = []
            for c in range(NCH):
                xd = axp - (iota_f + float(c * L)) * dxv
                xs.append(xd * xd)
            accs = [jnp.full((L,), ACC0, jnp.float32) for _ in range(NCH)]
            rowbase = off + p
            for r in range(KW):
                yd = ayp - float(r) * dxv
                yv = yd * yd
                for c in range(NCH):
                    driv = rows_v[rowbase + (r * NCH + c) * L, :]
                    accs[c] = jnp.minimum(accs[c], yv + driv * BIG)
            accp = jnp.minimum(jnp.minimum(xs[0] + accs[0], xs[1] + accs[1]),
                               xs[2] + accs[2])
            m = jnp.min(accp)
            return jnp.where(iota == pidx, m, accg)

        acc = lax.fori_loop(0, L, pbody, jnp.full((L,), ACC0, jnp.float32),
                            unroll=4)

        ratio = acc / pd2
        ibits = lax.bitcast_convert_type(ratio, jnp.int32)
        z = lax.bitcast_convert_type(jnp.int32(0x5F3759DF) - (ibits >> 1),
                                     jnp.float32)
        h = ratio * 0.5
        for _ in range(3):
            z = z * (1.5 - h * z * z)
        pen = jnp.where(ratio <= 1.0, 1.0 - ratio * z, 0.0)
        out_v[pl.ds(g * L, L)] = pen
        return carry

    lax.fori_loop(0, NG, group, 0)
    pltpu.sync_copy(out_v, out.at[pl.ds(wid * PPW, PPW)])


@jax.jit
def _coll_loss(table, trajf, vehf, mapi, dxa):
    mesh = plsc.VectorSubcoreMesh(core_axis_name="c", subcore_axis_name="s")
    f = functools.partial(
        pl.kernel,
        mesh=mesh,
        compiler_params=pltpu.CompilerParams(
            needs_layout_passes=False, use_tc_tiling_on_sc=False,
            skip_device_barrier=True),
        out_type=jax.ShapeDtypeStruct((NT,), jnp.float32),
        scratch_types=[
            pltpu.VMEM((2 * NROW * L, L), jnp.float32),
            pltpu.VMEM((PPW * 4,), jnp.float32),
            pltpu.VMEM((NA * 2,), jnp.float32),
            pltpu.VMEM((NA,), jnp.int32),
            pltpu.VMEM((L,), jnp.float32),
            pltpu.VMEM((PPW,), jnp.float32),
            pltpu.VMEM((2 * L,), jnp.float32),
            pltpu.SemaphoreType.DMA,
            pltpu.SemaphoreType.DMA,
        ],
    )(_body)
    return f(table, trajf, vehf, mapi, dxa)


def kernel(traj, veh_att, drivable_raster, mapixes, dx):
    table = (drivable_raster
             .reshape(M, H // 8, 8, W // 128, 128)
             .transpose(0, 1, 3, 2, 4)
             .reshape(TROWS, L))
    trajf = traj.reshape(NT * 4)
    vehf = veh_att.reshape(NA * 2)
    dxa = jnp.full((L,), dx, dtype=jnp.float32)
    pen = _coll_loss(table, trajf, vehf, mapixes.astype(jnp.int32), dxa)
    return pen.reshape(NA, T)


# speedup vs baseline: 1.0204x; 1.0204x over previous
import functools

import jax
import jax.numpy as jnp
from jax import lax
from jax.experimental import pallas as pl
from jax.experimental.pallas import tpu as pltpu
from jax.experimental.pallas import tpu_sc as plsc

NA, T, M, H, W = 512, 20, 4, 1024, 1024
R = 12
KW = 2 * R + 1
NT = NA * T
NC, NS, L = 2, 16, 16
NWORK = NC * NS
PPW = NT // NWORK
NG = PPW // L
NCH = 3
NROW = KW * NCH
TROWS = M * H * W // L
BIG = 1e9
ACC0 = 1e12


def _body(table, trajf, vehf, mapi, dxa, out,
          rows_v, traj_v, veh_v, map_v, dx_v, out_v, grp_v, sem0, sem1):
    wid = lax.axis_index("s") * NC + lax.axis_index("c")
    iota = lax.iota(jnp.int32, L)

    cp1 = pltpu.async_copy(trajf.at[pl.ds(wid * (PPW * 4), PPW * 4)], traj_v,
                           sem0)
    cp2 = pltpu.async_copy(vehf.at[:], veh_v, sem0)
    cp3 = pltpu.async_copy(mapi.at[:], map_v, sem0)
    cp4 = pltpu.async_copy(dxa.at[:], dx_v, sem0)
    cp1.wait()
    cp2.wait()
    cp3.wait()
    cp4.wait()
    dxv = dx_v[...]

    def point_params(g):
        lp = g * L + iota
        px = plsc.load_gather(traj_v, [lp * 4])
        py = plsc.load_gather(traj_v, [lp * 4 + 1])
        gp = wid * PPW + lp
        aid = gp // T
        ln = plsc.load_gather(veh_v, [aid * 2])
        wd = plsc.load_gather(veh_v, [aid * 2 + 1])
        mapf = plsc.load_gather(map_v, [aid])
        pd2 = (ln * ln + wd * wd) * 0.25

        pixx = (px / dxv).astype(jnp.int32)
        pixy = (py / dxv).astype(jnp.int32)
        cx0 = pixx - R
        chunk = cx0 >> 4
        s = cx0 & 15
        y0 = pixy - R
        mb = mapf * (H * W // L)
        cps = []
        for c in range(NCH):
            ch = chunk + c
            cps.append(mb + ((ch >> 3) << 6) + (ch & 7))
        fx = px - (pixx.astype(jnp.float32) + 0.5) * dxv
        fy = py - (pixy.astype(jnp.float32) + 0.5) * dxv
        ax = fx + (s + R).astype(jnp.float32) * dxv
        ay = fy + jnp.float32(R) * dxv
        return (y0, cps), ax, ay, pd2

    def fire_group(g, par, base):
        off = par * NROW * L

        y0, cps = base

        def fire(oy, _):
            y = y0 + oy
            ypart = ((y >> 3) << 9) + ((y & 7) << 3)
            for c in range(NCH):
                k = oy * NCH + c
                pltpu.async_copy(table.at[cps[c] + ypart],
                                 rows_v.at[pl.ds(off + k * L, L)],
                                 sem0 if par == 0 else sem1)
            return 0
        lax.fori_loop(0, KW, fire, 0)

    def drain_group(par):
        off = par * NROW * L
        pltpu.make_async_copy(table.at[pl.ds(0, NROW * L)],
                              rows_v.at[pl.ds(off, NROW * L)],
                              sem0 if par == 0 else sem1).wait()

    base0, _, _, _ = point_params(0)
    fire_group(0, 0, base0)

    def group(g, carry):
        par = lax.rem(g, 2)
        base, ax, ay, pd2 = point_params(g)

        @pl.when(g + 1 < NG)
        def _():
            nbase, _, _, _ = point_params(g + 1)
            @pl.when(par == 0)
            def _():
                fire_group(g + 1, 1, nbase)

            @pl.when(par == 1)
            def _():
                fire_group(g + 1, 0, nbase)

        @pl.when(par == 0)
        def _():
            drain_group(0)

        @pl.when(par == 1)
        def _():
            drain_group(1)

        off = par * (NROW * L)
        iota_f = iota.astype(jnp.float32)
        grp_v[pl.ds(0, L)] = ax
        grp_v[pl.ds(L, L)] = ay

        def pbody(p, accg):
            pidx = jnp.full((L,), p, jnp.int32)
            axp = plsc.load_gather(grp_v, [pidx])
            ayp = plsc.load_gather(grp_v, [pidx + L])
            xs = []
            for c in range(NCH):
                xd = axp - (iota_f + float(c * L)) * dxv
                xs.append(xd * xd)
            accs = [jnp.full((L,), ACC0, jnp.float32) for _ in range(NCH)]
            rowbase = off + p
            for r in range(KW):
                yd = ayp - float(r) * dxv
                yv = yd * yd
                for c in range(NCH):
                    driv = rows_v[rowbase + (r * NCH + c) * L, :]
                    accs[c] = jnp.minimum(accs[c], yv + driv * BIG)
            accp = jnp.minimum(jnp.minimum(xs[0] + accs[0], xs[1] + accs[1]),
                               xs[2] + accs[2])
            m = jnp.min(accp)
            return jnp.where(iota == pidx, m, accg)

        acc = lax.fori_loop(0, L, pbody, jnp.full((L,), ACC0, jnp.float32),
                            unroll=4)

        ratio = acc / pd2
        ibits = lax.bitcast_convert_type(ratio, jnp.int32)
        z = lax.bitcast_convert_type(jnp.int32(0x5F3759DF) - (ibits >> 1),
                                     jnp.float32)
        h = ratio * 0.5
        for _ in range(3):
            z = z * (1.5 - h * z * z)
        pen = jnp.where(ratio <= 1.0, 1.0 - ratio * z, 0.0)
        out_v[pl.ds(g * L, L)] = pen
        return carry

    lax.fori_loop(0, NG, group, 0)
    pltpu.sync_copy(out_v, out.at[pl.ds(wid * PPW, PPW)])


@jax.jit
def _coll_loss(table, trajf, vehf, mapi, dxa):
    mesh = plsc.VectorSubcoreMesh(core_axis_name="c", subcore_axis_name="s")
    f = functools.partial(
        pl.kernel,
        mesh=mesh,
        compiler_params=pltpu.CompilerParams(
            needs_layout_passes=False, use_tc_tiling_on_sc=False),
        out_type=jax.ShapeDtypeStruct((NT,), jnp.float32),
        scratch_types=[
            pltpu.VMEM((2 * NROW * L, L), jnp.float32),
            pltpu.VMEM((PPW * 4,), jnp.float32),
            pltpu.VMEM((NA * 2,), jnp.float32),
            pltpu.VMEM((NA,), jnp.int32),
            pltpu.VMEM((L,), jnp.float32),
            pltpu.VMEM((PPW,), jnp.float32),
            pltpu.VMEM((2 * L,), jnp.float32),
            pltpu.SemaphoreType.DMA,
            pltpu.SemaphoreType.DMA,
        ],
    )(_body)
    return f(table, trajf, vehf, mapi, dxa)


def kernel(traj, veh_att, drivable_raster, mapixes, dx):
    table = (drivable_raster
             .reshape(M, H // 8, 8, W // 128, 128)
             .transpose(0, 1, 3, 2, 4)
             .reshape(TROWS, L))
    trajf = traj.reshape(NT * 4)
    vehf = veh_att.reshape(NA * 2)
    dxa = jnp.full((L,), dx, dtype=jnp.float32)
    pen = _coll_loss(table, trajf, vehf, mapixes.astype(jnp.int32), dxa)
    return pen.reshape(NA, T)
